# baseline (device time: 533266 ns/iter reference)
import jax
import jax.numpy as jnp
from jax import lax
from jax.experimental import pallas as pl
from jax.experimental.pallas import tpu as pltpu


def kernel(x, W):
    m, k = x.shape
    _, v_loc = W.shape

    NB = 1024
    NCHUNK = v_loc // NB
    NSLOT = 4
    HOLD = 4

    def body(
        x_ref,
        w_ref,
        out_ref,
        full_ref,
        lblk,
        rchunk,
        nbuf,
        m_run,
        s_run,
        stats_snd,
        stats_rcv,
        copy_sems,
        send_sems,
        recv_sems,
        st_send_sem,
        st_recv_sem,
        rload_sem,
        owrite_sem,
    ):
        j = pl.program_id(0)
        my_x = lax.axis_index("x")
        my_y = lax.axis_index("y")

        def chunk_copy(jj, sem):
            return pltpu.make_async_copy(
                lblk.at[lax.rem(jj, NSLOT)],
                full_ref.at[my_x, :, pl.ds(jj * NB, NB)],
                sem,
            )

        def chunk_rdma(jj, send_sem, recv_sem):
            return pltpu.make_async_remote_copy(
                src_ref=lblk.at[lax.rem(jj, NSLOT)],
                dst_ref=full_ref.at[my_x, :, pl.ds(jj * NB, NB)],
                send_sem=send_sem,
                recv_sem=recv_sem,
                device_id=(1 - my_x, my_y),
                device_id_type=pl.DeviceIdType.MESH,
            )

        def chunk_rdma_hbm(jj, send_sem, recv_sem):
            return pltpu.make_async_remote_copy(
                src_ref=full_ref.at[my_x, :, pl.ds(jj * NB, NB)],
                dst_ref=full_ref.at[my_x, :, pl.ds(jj * NB, NB)],
                send_sem=send_sem,
                recv_sem=recv_sem,
                device_id=(1 - my_x, my_y),
                device_id_type=pl.DeviceIdType.MESH,
            )

        @pl.when(j >= NSLOT)
        def _():
            chunk_copy(j - NSLOT, copy_sems.at[j - NSLOT]).wait()

        @pl.when(jnp.logical_and(j >= NSLOT, j < HOLD + NSLOT))
        def _():
            chunk_rdma(
                j - NSLOT, send_sems.at[j - NSLOT], recv_sems.at[j - NSLOT]
            ).wait_send()

        slot = lax.rem(j, NSLOT)
        lblk[slot] = jnp.dot(
            x_ref[...].astype(jnp.bfloat16),
            w_ref[...].astype(jnp.bfloat16),
            preferred_element_type=jnp.float32,
        ).astype(jnp.bfloat16)

        chunk_copy(j, copy_sems.at[j]).start()

        @pl.when(j < HOLD)
        def _():
            chunk_rdma(j, send_sems.at[j], recv_sems.at[j]).start()

        @pl.when(j == 0)
        def _():
            m_run[...] = jnp.full((m, 1), -1e30, jnp.float32)
            s_run[...] = jnp.zeros((m, 1), jnp.float32)

        lf = lblk[slot].astype(jnp.float32)
        cm = lf.max(axis=1, keepdims=True)
        m_new = jnp.maximum(m_run[...], cm)
        s_run[...] = s_run[...] * jnp.exp(m_run[...] - m_new) + jnp.exp(
            lf - m_new
        ).sum(axis=1, keepdims=True)
        m_run[...] = m_new

        @pl.when(j == NCHUNK - 1)
        def _():
            for jj in range(NCHUNK - NSLOT, NCHUNK):
                chunk_copy(jj, copy_sems.at[jj]).wait()

            stats_snd[0] = m_run[...]
            stats_snd[1] = s_run[...]
            st = pltpu.make_async_remote_copy(
                src_ref=stats_snd,
                dst_ref=stats_rcv,
                send_sem=st_send_sem,
                recv_sem=st_recv_sem,
                device_id=(1 - my_x, my_y),
                device_id_type=pl.DeviceIdType.MESH,
            )
            st.start()
            for jj in range(HOLD, NCHUNK):
                chunk_rdma_hbm(jj, send_sems.at[jj], recv_sems.at[jj]).start()
            st.wait_recv()

            om = stats_rcv[0]
            os_ = stats_rcv[1]
            gm = jnp.maximum(m_run[...], om)
            gs = s_run[...] * jnp.exp(m_run[...] - gm) + os_ * jnp.exp(om - gm)
            inv = 1.0 / gs

            def normalize(half_idx, jj):
                ld = pltpu.make_async_copy(
                    full_ref.at[half_idx, :, pl.ds(jj * NB, NB)],
                    rchunk,
                    rload_sem,
                )
                ld.start()
                ld.wait()
                nbuf[...] = jnp.exp(rchunk[...].astype(jnp.float32) - gm) * inv
                col = half_idx * v_loc + jj * NB
                ow = pltpu.make_async_copy(
                    nbuf, out_ref.at[:, pl.ds(col, NB)], owrite_sem
                )
                ow.start()
                ow.wait()

            for jj in range(NCHUNK):
                normalize(my_x, jj)
            for jj in range(NCHUNK):
                chunk_rdma(jj, send_sems.at[jj], recv_sems.at[jj]).wait_recv()
                normalize(1 - my_x, jj)
            for jj in range(HOLD, NCHUNK):
                chunk_rdma_hbm(
                    jj, send_sems.at[jj], recv_sems.at[jj]
                ).wait_send()
            st.wait_send()

    out, _ = pl.pallas_call(
        body,
        grid=(NCHUNK,),
        in_specs=[
            pl.BlockSpec((m, k), lambda j: (0, 0)),
            pl.BlockSpec((k, NB), lambda j: (0, j)),
        ],
        out_specs=[
            pl.BlockSpec(memory_space=pl.ANY),
            pl.BlockSpec(memory_space=pl.ANY),
        ],
        out_shape=[
            jax.ShapeDtypeStruct((m, 2 * v_loc), jnp.float32),
            jax.ShapeDtypeStruct((2, m, v_loc), jnp.bfloat16),
        ],
        scratch_shapes=[
            pltpu.VMEM((NSLOT, m, NB), jnp.bfloat16),
            pltpu.VMEM((m, NB), jnp.bfloat16),
            pltpu.VMEM((m, NB), jnp.float32),
            pltpu.VMEM((m, 1), jnp.float32),
            pltpu.VMEM((m, 1), jnp.float32),
            pltpu.VMEM((2, m, 1), jnp.float32),
            pltpu.VMEM((2, m, 1), jnp.float32),
            pltpu.SemaphoreType.DMA((NCHUNK,)),
            pltpu.SemaphoreType.DMA((NCHUNK,)),
            pltpu.SemaphoreType.DMA((NCHUNK,)),
            pltpu.SemaphoreType.DMA,
            pltpu.SemaphoreType.DMA,
            pltpu.SemaphoreType.DMA,
            pltpu.SemaphoreType.DMA,
        ],
        compiler_params=pltpu.CompilerParams(
            has_side_effects=True,
            vmem_limit_bytes=60 * 1024 * 1024,
        ),
    )(x, W)
    return out


# device time: 484843 ns/iter; 1.0999x vs baseline; 1.0999x over previous
import jax
import jax.numpy as jnp
from jax import lax
from jax.experimental import pallas as pl
from jax.experimental.pallas import tpu as pltpu


def kernel(x, W):
    m, k = x.shape
    _, v_loc = W.shape

    NB = 1024
    NCHUNK = v_loc // NB
    NSLOT = 11
    HOLD = 5

    def body(
        x_ref,
        w_ref,
        out_ref,
        full_ref,
        lblk,
        rchunk,
        nbuf,
        stats_snd,
        stats_rcv,
        copy_sems,
        send_sems,
        recv_sems,
        st_send_sem,
        st_recv_sem,
        rload_sem,
        owrite_sem,
    ):
        j = pl.program_id(0)
        my_x = lax.axis_index("x")
        my_y = lax.axis_index("y")

        def chunk_copy(jj, sem):
            return pltpu.make_async_copy(
                lblk.at[lax.rem(jj, NSLOT)],
                full_ref.at[my_x, :, pl.ds(jj * NB, NB)],
                sem,
            )

        def chunk_rdma(jj, send_sem, recv_sem):
            return pltpu.make_async_remote_copy(
                src_ref=lblk.at[lax.rem(jj, NSLOT)],
                dst_ref=full_ref.at[my_x, :, pl.ds(jj * NB, NB)],
                send_sem=send_sem,
                recv_sem=recv_sem,
                device_id=(1 - my_x, my_y),
                device_id_type=pl.DeviceIdType.MESH,
            )

        @pl.when(j >= NSLOT)
        def _():
            chunk_copy(j - NSLOT, copy_sems.at[j - NSLOT]).wait()

        @pl.when(jnp.logical_and(j >= NSLOT, j < HOLD + NSLOT))
        def _():
            chunk_rdma(
                j - NSLOT, send_sems.at[j - NSLOT], recv_sems.at[j - NSLOT]
            ).wait_send()

        slot = lax.rem(j, NSLOT)
        lblk[slot] = jnp.dot(
            x_ref[...].astype(jnp.bfloat16),
            w_ref[...].astype(jnp.bfloat16),
            preferred_element_type=jnp.float32,
        ).astype(jnp.bfloat16)

        chunk_copy(j, copy_sems.at[j]).start()

        @pl.when(j < HOLD)
        def _():
            chunk_rdma(j, send_sems.at[j], recv_sems.at[j]).start()

        @pl.when(j == 0)
        def _():
            stats_snd[0] = jnp.full((m, 1), -1e30, jnp.float32)
            stats_snd[1] = jnp.zeros((m, 1), jnp.float32)

        lf = lblk[slot].astype(jnp.float32)
        cm = lf.max(axis=1, keepdims=True)
        m_prev = stats_snd[0]
        m_new = jnp.maximum(m_prev, cm)
        stats_snd[1] = stats_snd[1] * jnp.exp(m_prev - m_new) + jnp.exp(
            lf - m_new
        ).sum(axis=1, keepdims=True)
        stats_snd[0] = m_new

        @pl.when(j == NCHUNK - 1)
        def _():
            for jj in range(NCHUNK - NSLOT, NCHUNK):
                chunk_copy(jj, copy_sems.at[jj]).wait()

            st = pltpu.make_async_remote_copy(
                src_ref=stats_snd,
                dst_ref=stats_rcv,
                send_sem=st_send_sem,
                recv_sem=st_recv_sem,
                device_id=(1 - my_x, my_y),
                device_id_type=pl.DeviceIdType.MESH,
            )
            st.start()
            for jj in range(HOLD, NCHUNK):
                chunk_rdma(jj, send_sems.at[jj], recv_sems.at[jj]).start()
            st.wait_recv()

            om = stats_rcv[0]
            os_ = stats_rcv[1]
            gm = jnp.maximum(stats_snd[0], om)
            gs = stats_snd[1] * jnp.exp(stats_snd[0] - gm) + os_ * jnp.exp(
                om - gm
            )
            inv = 1.0 / gs

            def normalize(half_idx, jj):
                ld = pltpu.make_async_copy(
                    full_ref.at[half_idx, :, pl.ds(jj * NB, NB)],
                    rchunk,
                    rload_sem,
                )
                ld.start()
                ld.wait()
                nbuf[...] = jnp.exp(rchunk[...].astype(jnp.float32) - gm) * inv
                col = half_idx * v_loc + jj * NB
                ow = pltpu.make_async_copy(
                    nbuf, out_ref.at[:, pl.ds(col, NB)], owrite_sem
                )
                ow.start()
                ow.wait()

            for jj in range(NCHUNK):
                normalize(my_x, jj)
            for jj in range(NCHUNK):
                chunk_rdma(jj, send_sems.at[jj], recv_sems.at[jj]).wait_recv()
                normalize(1 - my_x, jj)
            for jj in range(HOLD, NCHUNK):
                chunk_rdma(jj, send_sems.at[jj], recv_sems.at[jj]).wait_send()
            st.wait_send()

    out, _ = pl.pallas_call(
        body,
        grid=(NCHUNK,),
        in_specs=[
            pl.BlockSpec((m, k), lambda j: (0, 0)),
            pl.BlockSpec((k, NB), lambda j: (0, j)),
        ],
        out_specs=[
            pl.BlockSpec(memory_space=pl.ANY),
            pl.BlockSpec(memory_space=pl.ANY),
        ],
        out_shape=[
            jax.ShapeDtypeStruct((m, 2 * v_loc), jnp.float32),
            jax.ShapeDtypeStruct((2, m, v_loc), jnp.bfloat16),
        ],
        scratch_shapes=[
            pltpu.VMEM((NSLOT, m, NB), jnp.bfloat16),
            pltpu.VMEM((m, NB), jnp.bfloat16),
            pltpu.VMEM((m, NB), jnp.float32),
            pltpu.VMEM((2, m, 1), jnp.float32),
            pltpu.VMEM((2, m, 1), jnp.float32),
            pltpu.SemaphoreType.DMA((NCHUNK,)),
            pltpu.SemaphoreType.DMA((NCHUNK,)),
            pltpu.SemaphoreType.DMA((NCHUNK,)),
            pltpu.SemaphoreType.DMA,
            pltpu.SemaphoreType.DMA,
            pltpu.SemaphoreType.DMA,
            pltpu.SemaphoreType.DMA,
        ],
        compiler_params=pltpu.CompilerParams(
            has_side_effects=True,
            vmem_limit_bytes=63 * 1024 * 1024,
        ),
    )(x, W)
    return out
